# Initial kernel scaffold; baseline (speedup 1.0000x reference)
import jax
import jax.numpy as jnp
from jax.experimental import pallas as pl


def kernel(x_local, top2_exp_id, top2_weight, W1_0, b1_0, W2_0, b2_0, W1_1, b1_1, W2_1, b2_1):
    raise NotImplementedError("write your pallas kernel here")


# speedup vs baseline: 1.3846x; 1.3846x over previous
import functools

import jax
import jax.numpy as jnp
from jax.experimental import pallas as pl

N_TOK = 4096
D_MODEL = 2048
D_FF = 8192

N_HALF = 4
E = 2
BF = 1024
NJ = D_FF // BF
TOK = N_TOK // N_HALF
BT = 256


def _ffn_moe_kernel(eid_ref, w_ref, x_ref, W1_ref, W2_ref, b1_ref, b2_ref,
                    out_ref):
    e = pl.program_id(1)
    j = pl.program_id(2)

    @pl.when(jnp.logical_and(e == 0, j == 0))
    def _init():
        out_ref[...] = jnp.zeros_like(out_ref)

    eid0 = eid_ref[:, 0:1]
    eid1 = eid_ref[:, 1:2]
    s = (jnp.where(eid0 == e, w_ref[:, 0:1], 0.0)
         + jnp.where(eid1 == e, w_ref[:, 1:2], 0.0))

    W1 = W1_ref[0]
    W2 = W2_ref[0]
    b1 = b1_ref[0, 0]
    b2 = b2_ref[0]

    for t in range(TOK // BT):
        rows = slice(t * BT, (t + 1) * BT)
        xt = x_ref[rows, :]
        h = jax.lax.dot_general(xt, W1, (((1,), (0,)), ((), ())),
                                preferred_element_type=jnp.float32)
        h = h + b1
        h = 0.5 * h * (1.0 + jax.lax.erf(h * 0.7071067811865476))
        part = jax.lax.dot_general(h.astype(jnp.bfloat16), W2,
                                   (((1,), (0,)), ((), ())),
                                   preferred_element_type=jnp.float32)
        st = s[rows, :]
        contrib = st * part

        @pl.when(j == 0)
        def _with_bias():
            out_ref[rows, :] += contrib + st * b2

        @pl.when(j != 0)
        def _no_bias():
            out_ref[rows, :] += contrib


def kernel(x_local, top2_exp_id, top2_weight, W1_0, b1_0, W2_0, b2_0,
           W1_1, b1_1, W2_1, b2_1):
    x_bf = x_local.astype(jnp.bfloat16)
    W1s = jnp.stack([W1_0, W1_1]).astype(jnp.bfloat16)
    W2s = jnp.stack([W2_0, W2_1]).astype(jnp.bfloat16)
    b1s = jnp.stack([b1_0, b1_1]).reshape(E, NJ, 1, BF)
    b2s = jnp.stack([b2_0, b2_1]).reshape(E, 1, D_MODEL)

    grid = (N_HALF, E, NJ)

    out = pl.pallas_call(
        _ffn_moe_kernel,
        grid=grid,
        in_specs=[
            pl.BlockSpec((TOK, 2), lambda h, e, j: (h, 0)),
            pl.BlockSpec((TOK, 2), lambda h, e, j: (h, 0)),
            pl.BlockSpec((TOK, D_MODEL), lambda h, e, j: (h, 0)),
            pl.BlockSpec((1, D_MODEL, BF), lambda h, e, j: (e, 0, j)),
            pl.BlockSpec((1, BF, D_MODEL), lambda h, e, j: (e, j, 0)),
            pl.BlockSpec((1, 1, 1, BF), lambda h, e, j: (e, j, 0, 0)),
            pl.BlockSpec((1, 1, D_MODEL), lambda h, e, j: (e, 0, 0)),
        ],
        out_specs=pl.BlockSpec((TOK, D_MODEL), lambda h, e, j: (h, 0)),
        out_shape=jax.ShapeDtypeStruct((N_TOK, D_MODEL), jnp.float32),
    )(top2_exp_id, top2_weight, x_bf, W1s, W2s, b1s, b2s)
    return out


# speedup vs baseline: 1.4368x; 1.0378x over previous
import functools

import jax
import jax.numpy as jnp
from jax.experimental import pallas as pl

N_TOK = 4096
D_MODEL = 2048
D_FF = 8192

N_HALF = 4
E = 2
BF = 1024
NJ = D_FF // BF
TOK = N_TOK // N_HALF
BT = 1024


def _ffn_moe_kernel(eid_ref, w_ref, x_ref, W1_ref, W2_ref, b1_ref, b2_ref,
                    out_ref):
    e = pl.program_id(1)
    j = pl.program_id(2)

    @pl.when(jnp.logical_and(e == 0, j == 0))
    def _init():
        out_ref[...] = jnp.zeros_like(out_ref)

    eid0 = eid_ref[:, 0:1]
    eid1 = eid_ref[:, 1:2]
    s = (jnp.where(eid0 == e, w_ref[:, 0:1], 0.0)
         + jnp.where(eid1 == e, w_ref[:, 1:2], 0.0))

    W1 = W1_ref[0]
    W2 = W2_ref[0]
    b1 = b1_ref[0, 0]
    b2 = b2_ref[0]

    for t in range(TOK // BT):
        rows = slice(t * BT, (t + 1) * BT)
        xt = x_ref[rows, :]
        h = jax.lax.dot_general(xt, W1, (((1,), (0,)), ((), ())),
                                preferred_element_type=jnp.float32)
        h = h + b1
        h = 0.5 * h * (1.0 + jax.lax.erf(h * 0.7071067811865476))
        part = jax.lax.dot_general(h.astype(jnp.bfloat16), W2,
                                   (((1,), (0,)), ((), ())),
                                   preferred_element_type=jnp.float32)
        st = s[rows, :]
        contrib = st * part

        @pl.when(j == 0)
        def _with_bias():
            out_ref[rows, :] += contrib + st * b2

        @pl.when(j != 0)
        def _no_bias():
            out_ref[rows, :] += contrib


def kernel(x_local, top2_exp_id, top2_weight, W1_0, b1_0, W2_0, b2_0,
           W1_1, b1_1, W2_1, b2_1):
    x_bf = x_local.astype(jnp.bfloat16)
    W1s = jnp.stack([W1_0, W1_1]).astype(jnp.bfloat16)
    W2s = jnp.stack([W2_0, W2_1]).astype(jnp.bfloat16)
    b1s = jnp.stack([b1_0, b1_1]).reshape(E, NJ, 1, BF)
    b2s = jnp.stack([b2_0, b2_1]).reshape(E, 1, D_MODEL)

    grid = (N_HALF, E, NJ)

    out = pl.pallas_call(
        _ffn_moe_kernel,
        grid=grid,
        in_specs=[
            pl.BlockSpec((TOK, 2), lambda h, e, j: (h, 0)),
            pl.BlockSpec((TOK, 2), lambda h, e, j: (h, 0)),
            pl.BlockSpec((TOK, D_MODEL), lambda h, e, j: (h, 0)),
            pl.BlockSpec((1, D_MODEL, BF), lambda h, e, j: (e, 0, j)),
            pl.BlockSpec((1, BF, D_MODEL), lambda h, e, j: (e, j, 0)),
            pl.BlockSpec((1, 1, 1, BF), lambda h, e, j: (e, j, 0, 0)),
            pl.BlockSpec((1, 1, D_MODEL), lambda h, e, j: (e, 0, 0)),
        ],
        out_specs=pl.BlockSpec((TOK, D_MODEL), lambda h, e, j: (h, 0)),
        out_shape=jax.ShapeDtypeStruct((N_TOK, D_MODEL), jnp.float32),
    )(top2_exp_id, top2_weight, x_bf, W1s, W2s, b1s, b2s)
    return out
